# baseline (speedup 1.0000x reference)
import functools

import jax
import jax.numpy as jnp
from jax import lax
from jax.experimental import pallas as pl
from jax.experimental.pallas import tpu as pltpu
from jax.experimental.pallas import tpu_sc as plsc

H = 128
N_NODES = 50000
_ARITY = (1, 2, 3)
_E = (200000, 200000, 150000)
_EPAD_G = (204800, 204800, 159744)
_EPAD_S = (204800, 204800, 155648)
_SINK = N_NODES
_ACC_ROWS = N_NODES + 48

_MESH = dict(core_axis_name="c", subcore_axis_name="s", num_cores=2,
             num_subcores=16)


_IDXB = tuple(e // 32 for e in _EPAD_G)
_IDXB_OFF = (0, _IDXB[0], _IDXB[0] + _IDXB[1])
_IDXB_TOT = sum(_IDXB)


def _gather_sc(node_states16, idx, epad):
    mesh = plsc.VectorSubcoreMesh(**_MESH)

    @functools.partial(
        pl.kernel,
        out_type=jax.ShapeDtypeStruct((epad, H), jnp.bfloat16),
        mesh=mesh,
        scratch_types=[
            pltpu.VMEM((128,), jnp.int32),
            pltpu.VMEM((128, H), jnp.bfloat16),
            pltpu.SemaphoreType.DMA,
        ],
        compiler_params=pltpu.CompilerParams(use_tc_tiling_on_sc=False),
    )
    def k(ns_hbm, ih, gh, idx_v, rows_v, sem):
        w = lax.axis_index("s") * 2 + lax.axis_index("c")
        units = epad // (128 * 32)
        row0 = w * units

        def body(i, _):
            row = row0 + i
            pltpu.sync_copy(ih.at[pl.ds(row * 128, 128)], idx_v)
            pltpu.async_copy(ns_hbm.at[idx_v], rows_v, sem).wait()
            pltpu.sync_copy(rows_v, gh.at[pl.ds(row * 128, 128)])
            return 0

        lax.fori_loop(0, units, body, 0)

    return k(node_states16, idx)


def _mlp_tc(x, w1, b1, w2, b2, n_rows, block_rows):
    d = x.shape[1]
    grid = (n_rows // block_rows,)

    def body(x_ref, w1_ref, b1_ref, w2_ref, b2_ref, y_ref, mx_ref):
        h = jnp.maximum(
            jnp.dot(x_ref[...], w1_ref[...],
                    preferred_element_type=jnp.float32) + b1_ref[...], 0.0)
        y = jnp.dot(h.astype(jnp.bfloat16), w2_ref[...],
                    preferred_element_type=jnp.float32) + b2_ref[...]
        y_ref[...] = y.astype(jnp.bfloat16)

        @pl.when(pl.program_id(0) == 0)
        def _init():
            mx_ref[0, 0] = -jnp.inf

        mx_ref[0, 0] = jnp.maximum(mx_ref[0, 0], jnp.max(y))

    return pl.pallas_call(
        body,
        grid=grid,
        in_specs=[
            pl.BlockSpec((block_rows, d), lambda i: (i, 0)),
            pl.BlockSpec((d, d), lambda i: (0, 0)),
            pl.BlockSpec((1, d), lambda i: (0, 0)),
            pl.BlockSpec((d, d), lambda i: (0, 0)),
            pl.BlockSpec((1, d), lambda i: (0, 0)),
        ],
        out_specs=[
            pl.BlockSpec((block_rows, d), lambda i: (i, 0)),
            pl.BlockSpec(memory_space=pltpu.SMEM),
        ],
        out_shape=[
            jax.ShapeDtypeStruct((n_rows, d), jnp.bfloat16),
            jax.ShapeDtypeStruct((1, 1), jnp.float32),
        ],
    )(x, w1.astype(jnp.bfloat16), b1.reshape(1, d),
      w2.astype(jnp.bfloat16), b2.reshape(1, d))


def _exp_tc(y, m):
    e_rows = y.shape[0]
    bt = 1024
    grid = (pl.cdiv(e_rows, bt),)

    def body(y_ref, m_ref, u_ref):
        y32 = y_ref[...].astype(jnp.float32)
        u_ref[...] = jnp.exp(8.0 * (y32 - m_ref[0, 0])).astype(jnp.bfloat16)

    return pl.pallas_call(
        body,
        grid=grid,
        in_specs=[
            pl.BlockSpec((bt, H), lambda i: (i, 0)),
            pl.BlockSpec(memory_space=pltpu.SMEM),
        ],
        out_specs=pl.BlockSpec((bt, H), lambda i: (i, 0)),
        out_shape=jax.ShapeDtypeStruct((e_rows, H), jnp.bfloat16),
    )(y, m)


def _scatter_sc(u, s_idx, zeros, epad, e_rows):
    mesh = plsc.VectorSubcoreMesh(**_MESH)

    @functools.partial(
        pl.kernel,
        out_type=jax.ShapeDtypeStruct((2, N_NODES, 64), jnp.bfloat16),
        mesh=mesh,
        scratch_types=[
            pltpu.VMEM((2, 256), jnp.int32),
            pltpu.VMEM((2, 256), jnp.int32),
            pltpu.VMEM((2, 256, 64), jnp.bfloat16),
            pltpu.VMEM_SHARED((_ACC_ROWS, 64), jnp.bfloat16),
            pltpu.SemaphoreType.DMA,
            pltpu.SemaphoreType.DMA,
            pltpu.SemaphoreType.DMA,
            pltpu.SemaphoreType.DMA,
            pltpu.SemaphoreType.DMA,
            pltpu.SemaphoreType.DMA,
        ],
        compiler_params=pltpu.CompilerParams(use_tc_tiling_on_sc=False),
    )
    def k(vh, nh, zr, out, nidx, ridx, vals, acc_sh, sn0, sn1, sg0, sg1,
          ss0, ss1):
        c = lax.axis_index("c")
        s = lax.axis_index("s")
        iota = lax.iota(jnp.int32, 16)
        sn = (sn0, sn1)
        sg = (sg0, sg1)
        ss = (ss0, ss1)
        per_tile = epad // 16
        units = per_tile // 256
        e_base = s * per_tile
        e_max = e_rows - 1
        pltpu.sync_copy(zr.at[pl.ds(s * 3128, 3128)],
                        acc_sh.at[pl.ds(s * 3128, 3128)])
        plsc.subcore_barrier()

        def fire_nidx(u_, b):
            pltpu.async_copy(nh.at[pl.ds(e_base + u_ * 256, 256)],
                             nidx.at[b], sn[b])

        def fire_gather(u_, b):
            e0 = e_base + u_ * 256
            for t in range(16):
                e_vec = jnp.minimum(e0 + t * 16 + iota, e_max)
                ridx[b, pl.ds(t * 16, 16)] = e_vec * 2 + c
            pltpu.async_copy(vh.at[ridx.at[b]], vals.at[b], sg[b])

        def fire_scatter(b):
            pltpu.async_copy(vals.at[b], acc_sh.at[nidx.at[b]],
                             ss[b], add=True)

        def drain_nidx(b):
            pltpu.make_async_copy(nh.at[pl.ds(0, 256)], nidx.at[b],
                                  sn[b]).wait()

        def drain_gather(b):
            pltpu.make_async_copy(vh.at[pl.ds(0, 256)], vals.at[b],
                                  sg[b]).wait()

        def drain_scatter(b):
            pltpu.make_async_copy(vals.at[b], acc_sh.at[pl.ds(0, 256)],
                                  ss[b]).wait()

        def unit(u_, b):
            @pl.when(u_ >= 2)
            def _frees():
                drain_scatter(b)

            fire_nidx(u_, b)
            fire_gather(u_, b)

            @pl.when(u_ >= 1)
            def _flush():
                drain_gather(1 - b)
                drain_nidx(1 - b)
                fire_scatter(1 - b)

        def body(gi, _):
            unit(2 * gi, 0)
            unit(2 * gi + 1, 1)
            return 0

        lax.fori_loop(0, units // 2, body, 0)
        drain_gather(1)
        drain_nidx(1)
        fire_scatter(1)
        drain_scatter(0)
        drain_scatter(1)
        plsc.subcore_barrier()
        @pl.when(s < 15)
        def _wb_full():
            pltpu.sync_copy(acc_sh.at[pl.ds(s * 3128, 3128)],
                            out.at[c, pl.ds(s * 3128, 3128)])

        @pl.when(s == 15)
        def _wb_last():
            pltpu.sync_copy(acc_sh.at[pl.ds(15 * 3128, 3080)],
                            out.at[c, pl.ds(15 * 3128, 3080)])

    return k(u, s_idx, zeros)


def _update_tc(accs, scales, node_states, wu1a, wu1b, bu1, wu2, bu2, m):
    bt = 1000
    grid = (N_NODES // bt,)

    def body(*refs):
        (a00, a01, a10, a11, a20, a21,
         s0_ref, s1_ref, s2_ref, ns_ref, w1a_ref, w1b_ref, b1_ref, w2_ref,
         b2_ref, m_ref, o_ref) = refs
        ah = ((a00, a10, a20), (a01, a11, a21))
        sc = (s0_ref[0, 0], s1_ref[0, 0], s2_ref[0, 0])
        h = jnp.dot(ns_ref[...], w1b_ref[...],
                    preferred_element_type=jnp.float32) + b1_ref[...]
        for half in range(2):
            tot = (sc[0] * ah[half][0][0].astype(jnp.float32)
                   + sc[1] * ah[half][1][0].astype(jnp.float32)
                   + sc[2] * ah[half][2][0].astype(jnp.float32))
            t = jnp.log(tot + 1e-16) * 0.125 + m_ref[0, 0]
            h += jnp.dot(t.astype(jnp.bfloat16),
                         w1a_ref[pl.ds(half * 64, 64), :],
                         preferred_element_type=jnp.float32)
        h = jnp.maximum(h, 0.0)
        o_ref[...] = jnp.dot(h.astype(jnp.bfloat16), w2_ref[...],
                             preferred_element_type=jnp.float32) + b2_ref[...]

    hspecs = []
    for r in range(3):
        hspecs += [pl.BlockSpec((1, bt, 64), lambda i, cc=cc: (cc, i, 0))
                   for cc in range(2)]
    return pl.pallas_call(
        body,
        grid=grid,
        in_specs=hspecs + [
            pl.BlockSpec(memory_space=pltpu.SMEM),
            pl.BlockSpec(memory_space=pltpu.SMEM),
            pl.BlockSpec(memory_space=pltpu.SMEM),
            pl.BlockSpec((bt, H), lambda i: (i, 0)),
            pl.BlockSpec((H, 2 * H), lambda i: (0, 0)),
            pl.BlockSpec((H, 2 * H), lambda i: (0, 0)),
            pl.BlockSpec((1, 2 * H), lambda i: (0, 0)),
            pl.BlockSpec((2 * H, H), lambda i: (0, 0)),
            pl.BlockSpec((1, H), lambda i: (0, 0)),
            pl.BlockSpec(memory_space=pltpu.SMEM),
        ],
        out_specs=pl.BlockSpec((bt, H), lambda i: (i, 0)),
        out_shape=jax.ShapeDtypeStruct((N_NODES, H), jnp.float32),
    )(accs[0], accs[0], accs[1], accs[1], accs[2], accs[2],
      scales[0], scales[1], scales[2], node_states,
      wu1a.astype(jnp.bfloat16), wu1b, bu1.reshape(1, 2 * H),
      wu2.astype(jnp.bfloat16), bu2.reshape(1, H), m)


def _pad_idx(rel, epad, fill):
    pad = epad - rel.shape[0]
    return jnp.concatenate([rel, jnp.full((pad,), fill, dtype=jnp.int32)])


def kernel(node_states, relations_0, relations_1, relations_2, w0_1, b0_1,
           w0_2, b0_2, w1_1, b1_1, w1_2, b1_2, w2_1, b2_1, w2_2, b2_2, wu1,
           bu1, wu2, bu2):
    rels = (relations_0, relations_1, relations_2)
    params = ((w0_1, b0_1, w0_2, b0_2), (w1_1, b1_1, w1_2, b1_2),
              (w2_1, b2_1, w2_2, b2_2))

    ns16 = node_states.astype(jnp.bfloat16)
    g = [_gather_sc(ns16, _pad_idx(rels[r], _EPAD_G[r], 0), _EPAD_G[r])
         for r in range(3)]

    blocks = (4000, 2000, 2000)
    ys, ms = [], []
    for r in range(3):
        d = _ARITY[r] * H
        x = g[r].reshape(_EPAD_G[r] // _ARITY[r], d)
        y, m = _mlp_tc(x, params[r][0], params[r][1], params[r][2],
                       params[r][3], _E[r] // _ARITY[r], blocks[r])
        ys.append(y)
        ms.append(m)

    m_all = jnp.maximum(jnp.maximum(ms[0][0, 0], ms[1][0, 0]),
                        ms[2][0, 0]).reshape(1, 1)

    us = [_exp_tc(ys[r].reshape(_E[r], H), ms[r]) for r in range(3)]

    zeros = jnp.zeros((_ACC_ROWS, 64), dtype=jnp.bfloat16)
    accs = [
        _scatter_sc(us[r].reshape(2 * _E[r], 64),
                    _pad_idx(rels[r], _EPAD_S[r], _SINK), zeros,
                    _EPAD_S[r], _E[r])
        for r in range(3)
    ]
    scales = [jnp.exp(8.0 * (ms[r] - m_all)) for r in range(3)]

    return _update_tc(accs, scales, node_states, wu1[:H], wu1[H:], bu1, wu2,
                      bu2, m_all)


# speedup vs baseline: 1.0655x; 1.0655x over previous
import functools

import jax
import jax.numpy as jnp
from jax import lax
from jax.experimental import pallas as pl
from jax.experimental.pallas import tpu as pltpu
from jax.experimental.pallas import tpu_sc as plsc

H = 128
N_NODES = 50000
_ARITY = (1, 2, 3)
_E = (200000, 200000, 150000)
_EPAD_G = (204800, 204800, 159744)
_EPAD_S = (204800, 204800, 155648)
_SINK = N_NODES
_ACC_ROWS = N_NODES + 48

_MESH = dict(core_axis_name="c", subcore_axis_name="s", num_cores=2,
             num_subcores=16)


_IDXB = tuple(e // 32 for e in _EPAD_G)
_IDXB_OFF = (0, _IDXB[0], _IDXB[0] + _IDXB[1])
_IDXB_TOT = sum(_IDXB)


def _gather_sc(node_states16, idx, epad):
    mesh = plsc.VectorSubcoreMesh(**_MESH)

    @functools.partial(
        pl.kernel,
        out_type=jax.ShapeDtypeStruct((epad, H), jnp.bfloat16),
        mesh=mesh,
        scratch_types=[
            pltpu.VMEM((128,), jnp.int32),
            pltpu.VMEM((128, H), jnp.bfloat16),
            pltpu.SemaphoreType.DMA,
        ],
        compiler_params=pltpu.CompilerParams(use_tc_tiling_on_sc=False),
    )
    def k(ns_hbm, ih, gh, idx_v, rows_v, sem):
        w = lax.axis_index("s") * 2 + lax.axis_index("c")
        units = epad // (128 * 32)
        row0 = w * units

        def body(i, _):
            row = row0 + i
            pltpu.sync_copy(ih.at[pl.ds(row * 128, 128)], idx_v)
            pltpu.async_copy(ns_hbm.at[idx_v], rows_v, sem).wait()
            pltpu.sync_copy(rows_v, gh.at[pl.ds(row * 128, 128)])
            return 0

        lax.fori_loop(0, units, body, 0)

    return k(node_states16, idx)


def _mlp_tc(x, w1, b1, w2, b2, n_rows, block_rows):
    d = x.shape[1]
    grid = (n_rows // block_rows,)

    def body(x_ref, w1_ref, b1_ref, w2_ref, b2_ref, y_ref, mx_ref):
        h = jnp.maximum(
            jnp.dot(x_ref[...], w1_ref[...],
                    preferred_element_type=jnp.float32) + b1_ref[...], 0.0)
        y = jnp.dot(h.astype(jnp.bfloat16), w2_ref[...],
                    preferred_element_type=jnp.float32) + b2_ref[...]
        y_ref[...] = y.astype(jnp.bfloat16)

        @pl.when(pl.program_id(0) == 0)
        def _init():
            mx_ref[0, 0] = -jnp.inf

        mx_ref[0, 0] = jnp.maximum(mx_ref[0, 0], jnp.max(y))

    return pl.pallas_call(
        body,
        grid=grid,
        in_specs=[
            pl.BlockSpec((block_rows, d), lambda i: (i, 0)),
            pl.BlockSpec((d, d), lambda i: (0, 0)),
            pl.BlockSpec((1, d), lambda i: (0, 0)),
            pl.BlockSpec((d, d), lambda i: (0, 0)),
            pl.BlockSpec((1, d), lambda i: (0, 0)),
        ],
        out_specs=[
            pl.BlockSpec((block_rows, d), lambda i: (i, 0)),
            pl.BlockSpec(memory_space=pltpu.SMEM),
        ],
        out_shape=[
            jax.ShapeDtypeStruct((n_rows, d), jnp.bfloat16),
            jax.ShapeDtypeStruct((1, 1), jnp.float32),
        ],
    )(x, w1.astype(jnp.bfloat16), b1.reshape(1, d),
      w2.astype(jnp.bfloat16), b2.reshape(1, d))


def _exp_tc(y, m):
    e_rows = y.shape[0]
    bt = 1024
    grid = (pl.cdiv(e_rows, bt),)

    def body(y_ref, m_ref, u_ref):
        y32 = y_ref[...].astype(jnp.float32)
        u_ref[...] = jnp.exp(8.0 * (y32 - m_ref[0, 0]))

    return pl.pallas_call(
        body,
        grid=grid,
        in_specs=[
            pl.BlockSpec((bt, H), lambda i: (i, 0)),
            pl.BlockSpec(memory_space=pltpu.SMEM),
        ],
        out_specs=pl.BlockSpec((bt, H), lambda i: (i, 0)),
        out_shape=jax.ShapeDtypeStruct((e_rows, H), jnp.float32),
    )(y, m)


def _scatter_sc(u, s_idx, zeros, epad, e_rows):
    mesh = plsc.VectorSubcoreMesh(**_MESH)

    @functools.partial(
        pl.kernel,
        out_type=jax.ShapeDtypeStruct((4, N_NODES, 32), jnp.float32),
        mesh=mesh,
        scratch_types=[
            pltpu.VMEM((2, 256), jnp.int32),
            pltpu.VMEM((2, 256), jnp.int32),
            pltpu.VMEM((2, 256, 32), jnp.float32),
            pltpu.VMEM_SHARED((_ACC_ROWS, 32), jnp.float32),
            pltpu.SemaphoreType.DMA,
            pltpu.SemaphoreType.DMA,
            pltpu.SemaphoreType.DMA,
            pltpu.SemaphoreType.DMA,
            pltpu.SemaphoreType.DMA,
            pltpu.SemaphoreType.DMA,
        ],
        compiler_params=pltpu.CompilerParams(use_tc_tiling_on_sc=False),
    )
    def k(vh, nh, zr, out, nidx, ridx, vals, acc_sh, sn0, sn1, sg0, sg1,
          ss0, ss1):
        c = lax.axis_index("c")
        s = lax.axis_index("s")
        iota = lax.iota(jnp.int32, 16)
        sn = (sn0, sn1)
        sg = (sg0, sg1)
        ss = (ss0, ss1)
        per_tile = epad // 16
        units = per_tile // 256
        e_base = s * per_tile
        e_max = e_rows - 1
        for qi in range(2):
            q = 2 * c + qi
            pltpu.sync_copy(zr.at[pl.ds(s * 3128, 3128)],
                            acc_sh.at[pl.ds(s * 3128, 3128)])
            plsc.subcore_barrier()

            def fire_nidx(u_, b):
                pltpu.async_copy(nh.at[pl.ds(e_base + u_ * 256, 256)],
                                 nidx.at[b], sn[b])

            def fire_gather(u_, b, q=q):
                e0 = e_base + u_ * 256
                for t in range(16):
                    e_vec = jnp.minimum(e0 + t * 16 + iota, e_max)
                    ridx[b, pl.ds(t * 16, 16)] = e_vec * 4 + q
                pltpu.async_copy(vh.at[ridx.at[b]], vals.at[b], sg[b])

            def fire_scatter(b):
                pltpu.async_copy(vals.at[b], acc_sh.at[nidx.at[b]],
                                 ss[b], add=True)

            def drain_nidx(b):
                pltpu.make_async_copy(nh.at[pl.ds(0, 256)], nidx.at[b],
                                      sn[b]).wait()

            def drain_gather(b):
                pltpu.make_async_copy(vh.at[pl.ds(0, 256)], vals.at[b],
                                      sg[b]).wait()

            def drain_scatter(b):
                pltpu.make_async_copy(vals.at[b], acc_sh.at[pl.ds(0, 256)],
                                      ss[b]).wait()

            def unit(u_, b):
                @pl.when(u_ >= 2)
                def _frees():
                    drain_scatter(b)

                fire_nidx(u_, b)
                fire_gather(u_, b)

                @pl.when(u_ >= 1)
                def _flush():
                    drain_gather(1 - b)
                    drain_nidx(1 - b)
                    fire_scatter(1 - b)

            def body(gi, _):
                unit(2 * gi, 0)
                unit(2 * gi + 1, 1)
                return 0

            lax.fori_loop(0, units // 2, body, 0)
            drain_gather(1)
            drain_nidx(1)
            fire_scatter(1)
            drain_scatter(0)
            drain_scatter(1)
            plsc.subcore_barrier()
            @pl.when(s < 15)
            def _wb_full():
                pltpu.sync_copy(acc_sh.at[pl.ds(s * 3128, 3128)],
                                out.at[q, pl.ds(s * 3128, 3128)])

            @pl.when(s == 15)
            def _wb_last():
                pltpu.sync_copy(acc_sh.at[pl.ds(15 * 3128, 3080)],
                                out.at[q, pl.ds(15 * 3128, 3080)])

            plsc.subcore_barrier()

    return k(u, s_idx, zeros)


def _update_tc(accs, scales, node_states, wu1a, wu1b, bu1, wu2, bu2, m):
    bt = 1000
    grid = (N_NODES // bt,)

    def body(*refs):
        (a00, a01, a02, a03, a10, a11, a12, a13, a20, a21, a22, a23,
         s0_ref, s1_ref, s2_ref, ns_ref, w1a_ref, w1b_ref, b1_ref, w2_ref,
         b2_ref, m_ref, o_ref) = refs
        aq = ((a00, a10, a20), (a01, a11, a21), (a02, a12, a22),
              (a03, a13, a23))
        sc = (s0_ref[0, 0], s1_ref[0, 0], s2_ref[0, 0])
        h = jnp.dot(ns_ref[...], w1b_ref[...],
                    preferred_element_type=jnp.float32) + b1_ref[...]
        for q in range(4):
            tot = (sc[0] * aq[q][0][0] + sc[1] * aq[q][1][0]
                   + sc[2] * aq[q][2][0])
            t = jnp.log(tot + 1e-16) * 0.125 + m_ref[0, 0]
            h += jnp.dot(t, w1a_ref[pl.ds(q * 32, 32), :],
                         preferred_element_type=jnp.float32)
        h = jnp.maximum(h, 0.0)
        o_ref[...] = jnp.dot(h, w2_ref[...],
                             preferred_element_type=jnp.float32) + b2_ref[...]

    qspecs = []
    for r in range(3):
        qspecs += [pl.BlockSpec((1, bt, 32), lambda i, q=q: (q, i, 0))
                   for q in range(4)]
    return pl.pallas_call(
        body,
        grid=grid,
        in_specs=qspecs + [
            pl.BlockSpec(memory_space=pltpu.SMEM),
            pl.BlockSpec(memory_space=pltpu.SMEM),
            pl.BlockSpec(memory_space=pltpu.SMEM),
            pl.BlockSpec((bt, H), lambda i: (i, 0)),
            pl.BlockSpec((H, 2 * H), lambda i: (0, 0)),
            pl.BlockSpec((H, 2 * H), lambda i: (0, 0)),
            pl.BlockSpec((1, 2 * H), lambda i: (0, 0)),
            pl.BlockSpec((2 * H, H), lambda i: (0, 0)),
            pl.BlockSpec((1, H), lambda i: (0, 0)),
            pl.BlockSpec(memory_space=pltpu.SMEM),
        ],
        out_specs=pl.BlockSpec((bt, H), lambda i: (i, 0)),
        out_shape=jax.ShapeDtypeStruct((N_NODES, H), jnp.float32),
    )(accs[0], accs[0], accs[0], accs[0],
      accs[1], accs[1], accs[1], accs[1],
      accs[2], accs[2], accs[2], accs[2],
      scales[0], scales[1], scales[2], node_states, wu1a, wu1b,
      bu1.reshape(1, 2 * H), wu2, bu2.reshape(1, H), m)


def _pad_idx(rel, epad, fill):
    pad = epad - rel.shape[0]
    return jnp.concatenate([rel, jnp.full((pad,), fill, dtype=jnp.int32)])


def kernel(node_states, relations_0, relations_1, relations_2, w0_1, b0_1,
           w0_2, b0_2, w1_1, b1_1, w1_2, b1_2, w2_1, b2_1, w2_2, b2_2, wu1,
           bu1, wu2, bu2):
    rels = (relations_0, relations_1, relations_2)
    params = ((w0_1, b0_1, w0_2, b0_2), (w1_1, b1_1, w1_2, b1_2),
              (w2_1, b2_1, w2_2, b2_2))

    ns16 = node_states.astype(jnp.bfloat16)
    g = [_gather_sc(ns16, _pad_idx(rels[r], _EPAD_G[r], 0), _EPAD_G[r])
         for r in range(3)]

    blocks = (4000, 2000, 2000)
    ys, ms = [], []
    for r in range(3):
        d = _ARITY[r] * H
        x = g[r].reshape(_EPAD_G[r] // _ARITY[r], d)
        y, m = _mlp_tc(x, params[r][0], params[r][1], params[r][2],
                       params[r][3], _E[r] // _ARITY[r], blocks[r])
        ys.append(y)
        ms.append(m)

    m_all = jnp.maximum(jnp.maximum(ms[0][0, 0], ms[1][0, 0]),
                        ms[2][0, 0]).reshape(1, 1)

    us = [_exp_tc(ys[r].reshape(_E[r], H), ms[r]) for r in range(3)]

    zeros = jnp.zeros((_ACC_ROWS, 32), dtype=jnp.float32)
    accs = [
        _scatter_sc(us[r].reshape(4 * _E[r], 32),
                    _pad_idx(rels[r], _EPAD_S[r], _SINK), zeros,
                    _EPAD_S[r], _E[r])
        for r in range(3)
    ]
    scales = [jnp.exp(8.0 * (ms[r] - m_all)) for r in range(3)]

    return _update_tc(accs, scales, node_states, wu1[:H], wu1[H:], bu1, wu2,
                      bu2, m_all)


# speedup vs baseline: 1.0741x; 1.0081x over previous
import functools

import jax
import jax.numpy as jnp
from jax import lax
from jax.experimental import pallas as pl
from jax.experimental.pallas import tpu as pltpu
from jax.experimental.pallas import tpu_sc as plsc

H = 128
N_NODES = 50000
_ARITY = (1, 2, 3)
_E = (200000, 200000, 150000)
_EPAD_G = (204800, 204800, 159744)
_EPAD_S = (204800, 204800, 155648)
_SINK = N_NODES
_ACC_ROWS = N_NODES + 48

_MESH = dict(core_axis_name="c", subcore_axis_name="s", num_cores=2,
             num_subcores=16)


_IDXB = tuple(e // 32 for e in _EPAD_G)
_IDXB_OFF = (0, _IDXB[0], _IDXB[0] + _IDXB[1])
_IDXB_TOT = sum(_IDXB)


def _gather_sc(node_states16, idx, epad):
    mesh = plsc.VectorSubcoreMesh(**_MESH)
    units = epad // (128 * 32)

    @functools.partial(
        pl.kernel,
        out_type=jax.ShapeDtypeStruct((epad, H), jnp.bfloat16),
        mesh=mesh,
        scratch_types=[
            pltpu.VMEM((2, 128), jnp.int32),
            pltpu.VMEM((2, 128, H), jnp.bfloat16),
            pltpu.SemaphoreType.DMA,
            pltpu.SemaphoreType.DMA,
            pltpu.SemaphoreType.DMA,
        ],
        compiler_params=pltpu.CompilerParams(use_tc_tiling_on_sc=False),
    )
    def k(ns_hbm, ih, gh, idx_v, rows_v, sg0, sg1, ssem):
        w = lax.axis_index("s") * 2 + lax.axis_index("c")
        row0 = w * units
        sg = (sg0, sg1)

        def load_idx(u_, b):
            pltpu.sync_copy(ih.at[pl.ds((row0 + u_) * 128, 128)],
                            idx_v.at[b])

        def fire_gather(u_, b):
            pltpu.async_copy(ns_hbm.at[idx_v.at[b]], rows_v.at[b], sg[b])

        def drain_gather(b):
            pltpu.make_async_copy(gh.at[pl.ds(0, 128)], rows_v.at[b],
                                  sg[b]).wait()

        def store(u_, b):
            pltpu.sync_copy(rows_v.at[b],
                            gh.at[pl.ds((row0 + u_) * 128, 128)])

        def unit(u_, b):
            @pl.when(u_ + 1 < units)
            def _prefetch():
                load_idx(u_ + 1, 1 - b)
                fire_gather(u_ + 1, 1 - b)

            drain_gather(b)
            store(u_, b)

        load_idx(0, 0)
        fire_gather(0, 0)

        def body(gi, _):
            unit(2 * gi, 0)
            unit(2 * gi + 1, 1)
            return 0

        lax.fori_loop(0, units // 2, body, 0)
        if units % 2:
            unit(units - 1, 0)

    return k(node_states16, idx)


def _mlp_tc(x, w1, b1, w2, b2, n_rows, block_rows):
    d = x.shape[1]
    grid = (n_rows // block_rows,)

    def body(x_ref, w1_ref, b1_ref, w2_ref, b2_ref, y_ref, mx_ref):
        h = jnp.maximum(
            jnp.dot(x_ref[...], w1_ref[...],
                    preferred_element_type=jnp.float32) + b1_ref[...], 0.0)
        y = jnp.dot(h.astype(jnp.bfloat16), w2_ref[...],
                    preferred_element_type=jnp.float32) + b2_ref[...]
        y_ref[...] = y.astype(jnp.bfloat16)

        @pl.when(pl.program_id(0) == 0)
        def _init():
            mx_ref[0, 0] = -jnp.inf

        mx_ref[0, 0] = jnp.maximum(mx_ref[0, 0], jnp.max(y))

    return pl.pallas_call(
        body,
        grid=grid,
        in_specs=[
            pl.BlockSpec((block_rows, d), lambda i: (i, 0)),
            pl.BlockSpec((d, d), lambda i: (0, 0)),
            pl.BlockSpec((1, d), lambda i: (0, 0)),
            pl.BlockSpec((d, d), lambda i: (0, 0)),
            pl.BlockSpec((1, d), lambda i: (0, 0)),
        ],
        out_specs=[
            pl.BlockSpec((block_rows, d), lambda i: (i, 0)),
            pl.BlockSpec(memory_space=pltpu.SMEM),
        ],
        out_shape=[
            jax.ShapeDtypeStruct((n_rows, d), jnp.bfloat16),
            jax.ShapeDtypeStruct((1, 1), jnp.float32),
        ],
    )(x, w1.astype(jnp.bfloat16), b1.reshape(1, d),
      w2.astype(jnp.bfloat16), b2.reshape(1, d))


def _exp_tc(y, m):
    e_rows = y.shape[0]
    bt = 1024
    grid = (pl.cdiv(e_rows, bt),)

    def body(y_ref, m_ref, u_ref):
        y32 = y_ref[...].astype(jnp.float32)
        u_ref[...] = jnp.exp(8.0 * (y32 - m_ref[0, 0]))

    return pl.pallas_call(
        body,
        grid=grid,
        in_specs=[
            pl.BlockSpec((bt, H), lambda i: (i, 0)),
            pl.BlockSpec(memory_space=pltpu.SMEM),
        ],
        out_specs=pl.BlockSpec((bt, H), lambda i: (i, 0)),
        out_shape=jax.ShapeDtypeStruct((e_rows, H), jnp.float32),
    )(y, m)


def _scatter_sc(u, s_idx, zeros, epad, e_rows):
    mesh = plsc.VectorSubcoreMesh(**_MESH)

    @functools.partial(
        pl.kernel,
        out_type=jax.ShapeDtypeStruct((4, N_NODES, 32), jnp.float32),
        mesh=mesh,
        scratch_types=[
            pltpu.VMEM((2, 256), jnp.int32),
            pltpu.VMEM((2, 256), jnp.int32),
            pltpu.VMEM((2, 256, 32), jnp.float32),
            pltpu.VMEM_SHARED((_ACC_ROWS, 32), jnp.float32),
            pltpu.SemaphoreType.DMA,
            pltpu.SemaphoreType.DMA,
            pltpu.SemaphoreType.DMA,
            pltpu.SemaphoreType.DMA,
            pltpu.SemaphoreType.DMA,
            pltpu.SemaphoreType.DMA,
        ],
        compiler_params=pltpu.CompilerParams(use_tc_tiling_on_sc=False),
    )
    def k(vh, nh, zr, out, nidx, ridx, vals, acc_sh, sn0, sn1, sg0, sg1,
          ss0, ss1):
        c = lax.axis_index("c")
        s = lax.axis_index("s")
        iota = lax.iota(jnp.int32, 16)
        sn = (sn0, sn1)
        sg = (sg0, sg1)
        ss = (ss0, ss1)
        per_tile = epad // 16
        units = per_tile // 256
        e_base = s * per_tile
        e_max = e_rows - 1
        for qi in range(2):
            q = 2 * c + qi
            pltpu.sync_copy(zr.at[pl.ds(s * 3128, 3128)],
                            acc_sh.at[pl.ds(s * 3128, 3128)])
            plsc.subcore_barrier()

            def fire_nidx(u_, b):
                pltpu.async_copy(nh.at[pl.ds(e_base + u_ * 256, 256)],
                                 nidx.at[b], sn[b])

            def fire_gather(u_, b, q=q):
                e0 = e_base + u_ * 256
                for t in range(16):
                    e_vec = jnp.minimum(e0 + t * 16 + iota, e_max)
                    ridx[b, pl.ds(t * 16, 16)] = e_vec * 4 + q
                pltpu.async_copy(vh.at[ridx.at[b]], vals.at[b], sg[b])

            def fire_scatter(b):
                pltpu.async_copy(vals.at[b], acc_sh.at[nidx.at[b]],
                                 ss[b], add=True)

            def drain_nidx(b):
                pltpu.make_async_copy(nh.at[pl.ds(0, 256)], nidx.at[b],
                                      sn[b]).wait()

            def drain_gather(b):
                pltpu.make_async_copy(vh.at[pl.ds(0, 256)], vals.at[b],
                                      sg[b]).wait()

            def drain_scatter(b):
                pltpu.make_async_copy(vals.at[b], acc_sh.at[pl.ds(0, 256)],
                                      ss[b]).wait()

            def unit(u_, b):
                @pl.when(u_ >= 2)
                def _frees():
                    drain_scatter(b)

                fire_nidx(u_, b)
                fire_gather(u_, b)

                @pl.when(u_ >= 1)
                def _flush():
                    drain_gather(1 - b)
                    drain_nidx(1 - b)
                    fire_scatter(1 - b)

            def body(gi, _):
                unit(2 * gi, 0)
                unit(2 * gi + 1, 1)
                return 0

            lax.fori_loop(0, units // 2, body, 0)
            drain_gather(1)
            drain_nidx(1)
            fire_scatter(1)
            drain_scatter(0)
            drain_scatter(1)
            plsc.subcore_barrier()
            @pl.when(s < 15)
            def _wb_full():
                pltpu.sync_copy(acc_sh.at[pl.ds(s * 3128, 3128)],
                                out.at[q, pl.ds(s * 3128, 3128)])

            @pl.when(s == 15)
            def _wb_last():
                pltpu.sync_copy(acc_sh.at[pl.ds(15 * 3128, 3080)],
                                out.at[q, pl.ds(15 * 3128, 3080)])

            plsc.subcore_barrier()

    return k(u, s_idx, zeros)


def _update_tc(accs, scales, node_states, wu1a, wu1b, bu1, wu2, bu2, m):
    bt = 1000
    grid = (N_NODES // bt,)

    def body(*refs):
        (a00, a01, a02, a03, a10, a11, a12, a13, a20, a21, a22, a23,
         s0_ref, s1_ref, s2_ref, ns_ref, w1a_ref, w1b_ref, b1_ref, w2_ref,
         b2_ref, m_ref, o_ref) = refs
        aq = ((a00, a10, a20), (a01, a11, a21), (a02, a12, a22),
              (a03, a13, a23))
        sc = (s0_ref[0, 0], s1_ref[0, 0], s2_ref[0, 0])
        h = jnp.dot(ns_ref[...], w1b_ref[...],
                    preferred_element_type=jnp.float32) + b1_ref[...]
        for q in range(4):
            tot = (sc[0] * aq[q][0][0] + sc[1] * aq[q][1][0]
                   + sc[2] * aq[q][2][0])
            t = jnp.log(tot + 1e-16) * 0.125 + m_ref[0, 0]
            h += jnp.dot(t, w1a_ref[pl.ds(q * 32, 32), :],
                         preferred_element_type=jnp.float32)
        h = jnp.maximum(h, 0.0)
        o_ref[...] = jnp.dot(h, w2_ref[...],
                             preferred_element_type=jnp.float32) + b2_ref[...]

    qspecs = []
    for r in range(3):
        qspecs += [pl.BlockSpec((1, bt, 32), lambda i, q=q: (q, i, 0))
                   for q in range(4)]
    return pl.pallas_call(
        body,
        grid=grid,
        in_specs=qspecs + [
            pl.BlockSpec(memory_space=pltpu.SMEM),
            pl.BlockSpec(memory_space=pltpu.SMEM),
            pl.BlockSpec(memory_space=pltpu.SMEM),
            pl.BlockSpec((bt, H), lambda i: (i, 0)),
            pl.BlockSpec((H, 2 * H), lambda i: (0, 0)),
            pl.BlockSpec((H, 2 * H), lambda i: (0, 0)),
            pl.BlockSpec((1, 2 * H), lambda i: (0, 0)),
            pl.BlockSpec((2 * H, H), lambda i: (0, 0)),
            pl.BlockSpec((1, H), lambda i: (0, 0)),
            pl.BlockSpec(memory_space=pltpu.SMEM),
        ],
        out_specs=pl.BlockSpec((bt, H), lambda i: (i, 0)),
        out_shape=jax.ShapeDtypeStruct((N_NODES, H), jnp.float32),
    )(accs[0], accs[0], accs[0], accs[0],
      accs[1], accs[1], accs[1], accs[1],
      accs[2], accs[2], accs[2], accs[2],
      scales[0], scales[1], scales[2], node_states, wu1a, wu1b,
      bu1.reshape(1, 2 * H), wu2, bu2.reshape(1, H), m)


def _pad_idx(rel, epad, fill):
    pad = epad - rel.shape[0]
    return jnp.concatenate([rel, jnp.full((pad,), fill, dtype=jnp.int32)])


def kernel(node_states, relations_0, relations_1, relations_2, w0_1, b0_1,
           w0_2, b0_2, w1_1, b1_1, w1_2, b1_2, w2_1, b2_1, w2_2, b2_2, wu1,
           bu1, wu2, bu2):
    rels = (relations_0, relations_1, relations_2)
    params = ((w0_1, b0_1, w0_2, b0_2), (w1_1, b1_1, w1_2, b1_2),
              (w2_1, b2_1, w2_2, b2_2))

    ns16 = node_states.astype(jnp.bfloat16)
    g = [_gather_sc(ns16, _pad_idx(rels[r], _EPAD_G[r], 0), _EPAD_G[r])
         for r in range(3)]

    blocks = (4000, 2000, 2000)
    ys, ms = [], []
    for r in range(3):
        d = _ARITY[r] * H
        x = g[r].reshape(_EPAD_G[r] // _ARITY[r], d)
        y, m = _mlp_tc(x, params[r][0], params[r][1], params[r][2],
                       params[r][3], _E[r] // _ARITY[r], blocks[r])
        ys.append(y)
        ms.append(m)

    m_all = jnp.maximum(jnp.maximum(ms[0][0, 0], ms[1][0, 0]),
                        ms[2][0, 0]).reshape(1, 1)

    us = [_exp_tc(ys[r].reshape(_E[r], H), ms[r]) for r in range(3)]

    zeros = jnp.zeros((_ACC_ROWS, 32), dtype=jnp.float32)
    accs = [
        _scatter_sc(us[r].reshape(4 * _E[r], 32),
                    _pad_idx(rels[r], _EPAD_S[r], _SINK), zeros,
                    _EPAD_S[r], _E[r])
        for r in range(3)
    ]
    scales = [jnp.exp(8.0 * (ms[r] - m_all)) for r in range(3)]

    return _update_tc(accs, scales, node_states, wu1[:H], wu1[H:], bu1, wu2,
                      bu2, m_all)


# speedup vs baseline: 1.0870x; 1.0120x over previous
import functools

import jax
import jax.numpy as jnp
from jax import lax
from jax.experimental import pallas as pl
from jax.experimental.pallas import tpu as pltpu
from jax.experimental.pallas import tpu_sc as plsc

H = 128
N_NODES = 50000
_ARITY = (1, 2, 3)
_E = (200000, 200000, 150000)
_EPAD_G = (204800, 204800, 159744)
_EPAD_S = (204800, 204800, 155648)
_SINK = N_NODES
_ACC_ROWS = N_NODES + 48

_MESH = dict(core_axis_name="c", subcore_axis_name="s", num_cores=2,
             num_subcores=16)


_IDXB = tuple(e // 32 for e in _EPAD_G)
_IDXB_OFF = (0, _IDXB[0], _IDXB[0] + _IDXB[1])
_IDXB_TOT = sum(_IDXB)


def _gather_sc(node_states16, idx, epad):
    mesh = plsc.VectorSubcoreMesh(**_MESH)
    units = epad // (128 * 32)

    @functools.partial(
        pl.kernel,
        out_type=jax.ShapeDtypeStruct((epad, H), jnp.bfloat16),
        mesh=mesh,
        scratch_types=[
            pltpu.VMEM((2, 128), jnp.int32),
            pltpu.VMEM((2, 128, H), jnp.bfloat16),
            pltpu.SemaphoreType.DMA,
            pltpu.SemaphoreType.DMA,
            pltpu.SemaphoreType.DMA,
        ],
        compiler_params=pltpu.CompilerParams(use_tc_tiling_on_sc=False),
    )
    def k(ns_hbm, ih, gh, idx_v, rows_v, sg0, sg1, ssem):
        w = lax.axis_index("s") * 2 + lax.axis_index("c")
        row0 = w * units
        sg = (sg0, sg1)

        def load_idx(u_, b):
            pltpu.sync_copy(ih.at[pl.ds((row0 + u_) * 128, 128)],
                            idx_v.at[b])

        def fire_gather(u_, b):
            pltpu.async_copy(ns_hbm.at[idx_v.at[b]], rows_v.at[b], sg[b])

        def drain_gather(b):
            pltpu.make_async_copy(gh.at[pl.ds(0, 128)], rows_v.at[b],
                                  sg[b]).wait()

        def store(u_, b):
            pltpu.sync_copy(rows_v.at[b],
                            gh.at[pl.ds((row0 + u_) * 128, 128)])

        def unit(u_, b):
            @pl.when(u_ + 1 < units)
            def _prefetch():
                load_idx(u_ + 1, 1 - b)
                fire_gather(u_ + 1, 1 - b)

            drain_gather(b)
            store(u_, b)

        load_idx(0, 0)
        fire_gather(0, 0)

        def body(gi, _):
            unit(2 * gi, 0)
            unit(2 * gi + 1, 1)
            return 0

        lax.fori_loop(0, units // 2, body, 0)
        if units % 2:
            unit(units - 1, 0)

    return k(node_states16, idx)


def _mlp_tc(x, w1, b1, w2, b2, n_rows, block_rows):
    d = x.shape[1]
    grid = (n_rows // block_rows,)

    def body(x_ref, w1_ref, b1_ref, w2_ref, b2_ref, y_ref, mx_ref):
        h = jnp.maximum(
            jnp.dot(x_ref[...], w1_ref[...],
                    preferred_element_type=jnp.float32) + b1_ref[...], 0.0)
        y = jnp.dot(h.astype(jnp.bfloat16), w2_ref[...],
                    preferred_element_type=jnp.float32) + b2_ref[...]
        y_ref[...] = y.astype(jnp.bfloat16)

        @pl.when(pl.program_id(0) == 0)
        def _init():
            mx_ref[0, 0] = -jnp.inf

        mx_ref[0, 0] = jnp.maximum(mx_ref[0, 0], jnp.max(y))

    return pl.pallas_call(
        body,
        grid=grid,
        in_specs=[
            pl.BlockSpec((block_rows, d), lambda i: (i, 0)),
            pl.BlockSpec((d, d), lambda i: (0, 0)),
            pl.BlockSpec((1, d), lambda i: (0, 0)),
            pl.BlockSpec((d, d), lambda i: (0, 0)),
            pl.BlockSpec((1, d), lambda i: (0, 0)),
        ],
        out_specs=[
            pl.BlockSpec((block_rows, d), lambda i: (i, 0)),
            pl.BlockSpec(memory_space=pltpu.SMEM),
        ],
        out_shape=[
            jax.ShapeDtypeStruct((n_rows, d), jnp.bfloat16),
            jax.ShapeDtypeStruct((1, 1), jnp.float32),
        ],
    )(x, w1.astype(jnp.bfloat16), b1.reshape(1, d),
      w2.astype(jnp.bfloat16), b2.reshape(1, d))


def _exp_tc(y, m):
    e_rows = y.shape[0]
    bt = 1024
    grid = (pl.cdiv(e_rows, bt),)

    def body(y_ref, m_ref, u_ref):
        y32 = y_ref[...].astype(jnp.float32)
        u_ref[...] = jnp.exp(8.0 * (y32 - m_ref[0, 0]))

    return pl.pallas_call(
        body,
        grid=grid,
        in_specs=[
            pl.BlockSpec((bt, H), lambda i: (i, 0)),
            pl.BlockSpec(memory_space=pltpu.SMEM),
        ],
        out_specs=pl.BlockSpec((bt, H), lambda i: (i, 0)),
        out_shape=jax.ShapeDtypeStruct((e_rows, H), jnp.float32),
    )(y, m)


def _scatter_sc(u, s_idx, zeros, epad, e_rows):
    mesh = plsc.VectorSubcoreMesh(**_MESH)

    @functools.partial(
        pl.kernel,
        out_type=jax.ShapeDtypeStruct((4, N_NODES, 32), jnp.float32),
        mesh=mesh,
        scratch_types=[
            pltpu.VMEM((2, 256), jnp.int32),
            pltpu.VMEM((2, 256), jnp.int32),
            pltpu.VMEM((2, 256, 32), jnp.float32),
            pltpu.VMEM_SHARED((_ACC_ROWS, 32), jnp.float32),
            pltpu.SemaphoreType.DMA,
            pltpu.SemaphoreType.DMA,
            pltpu.SemaphoreType.DMA,
            pltpu.SemaphoreType.DMA,
            pltpu.SemaphoreType.DMA,
            pltpu.SemaphoreType.DMA,
        ],
        compiler_params=pltpu.CompilerParams(use_tc_tiling_on_sc=False),
    )
    def k(vh, nh, zr, out, nidx, ridx, vals, acc_sh, sn0, sn1, sg0, sg1,
          ss0, ss1):
        c = lax.axis_index("c")
        s = lax.axis_index("s")
        iota = lax.iota(jnp.int32, 16)
        sn = (sn0, sn1)
        sg = (sg0, sg1)
        ss = (ss0, ss1)
        per_tile = epad // 16
        units = per_tile // 256
        e_base = s * per_tile
        e_max = e_rows - 1
        for qi in range(2):
            q = 2 * c + qi
            pltpu.sync_copy(zr.at[pl.ds(s * 3128, 3128)],
                            acc_sh.at[pl.ds(s * 3128, 3128)])
            plsc.subcore_barrier()

            def fire_nidx(u_, b):
                pltpu.async_copy(nh.at[pl.ds(e_base + u_ * 256, 256)],
                                 nidx.at[b], sn[b])

            def fire_gather(u_, b, q=q):
                e0 = e_base + u_ * 256
                for t in range(16):
                    e_vec = jnp.minimum(e0 + t * 16 + iota, e_max)
                    ridx[b, pl.ds(t * 16, 16)] = e_vec * 4 + q
                pltpu.async_copy(vh.at[ridx.at[b]], vals.at[b], sg[b])

            def fire_scatter(b):
                pltpu.async_copy(vals.at[b], acc_sh.at[nidx.at[b]],
                                 ss[b], add=True)

            def drain_nidx(b):
                pltpu.make_async_copy(nh.at[pl.ds(0, 256)], nidx.at[b],
                                      sn[b]).wait()

            def drain_gather(b):
                pltpu.make_async_copy(vh.at[pl.ds(0, 256)], vals.at[b],
                                      sg[b]).wait()

            def drain_scatter(b):
                pltpu.make_async_copy(vals.at[b], acc_sh.at[pl.ds(0, 256)],
                                      ss[b]).wait()

            def unit(u_, b):
                @pl.when(u_ >= 2)
                def _frees():
                    drain_scatter(b)

                fire_nidx(u_, b)
                fire_gather(u_, b)

                @pl.when(u_ >= 1)
                def _flush():
                    drain_gather(1 - b)
                    drain_nidx(1 - b)
                    fire_scatter(1 - b)

            def body(gi, _):
                unit(2 * gi, 0)
                unit(2 * gi + 1, 1)
                return 0

            lax.fori_loop(0, units // 2, body, 0)
            drain_gather(1)
            drain_nidx(1)
            fire_scatter(1)
            drain_scatter(0)
            drain_scatter(1)
            plsc.subcore_barrier()
            @pl.when(s < 15)
            def _wb_full():
                pltpu.sync_copy(acc_sh.at[pl.ds(s * 3128, 3128)],
                                out.at[q, pl.ds(s * 3128, 3128)])

            @pl.when(s == 15)
            def _wb_last():
                pltpu.sync_copy(acc_sh.at[pl.ds(15 * 3128, 3080)],
                                out.at[q, pl.ds(15 * 3128, 3080)])

            plsc.subcore_barrier()

    return k(u, s_idx, zeros)


def _update_tc(accs, scales, node_states, wu1a, wu1b, bu1, wu2, bu2, m):
    bt = 1000
    grid = (N_NODES // bt,)

    def body(*refs):
        (a00, a01, a02, a03, a10, a11, a12, a13, a20, a21, a22, a23,
         s0_ref, s1_ref, s2_ref, ns_ref, w1a_ref, w1b_ref, b1_ref, w2_ref,
         b2_ref, m_ref, o_ref) = refs
        aq = ((a00, a10, a20), (a01, a11, a21), (a02, a12, a22),
              (a03, a13, a23))
        sc = (s0_ref[0, 0], s1_ref[0, 0], s2_ref[0, 0])
        h = jnp.dot(ns_ref[...].astype(jnp.bfloat16), w1b_ref[...],
                    preferred_element_type=jnp.float32) + b1_ref[...]
        for q in range(4):
            tot = (sc[0] * aq[q][0][0] + sc[1] * aq[q][1][0]
                   + sc[2] * aq[q][2][0])
            t = jnp.log(tot + 1e-16) * 0.125 + m_ref[0, 0]
            h += jnp.dot(t.astype(jnp.bfloat16),
                         w1a_ref[pl.ds(q * 32, 32), :],
                         preferred_element_type=jnp.float32)
        h = jnp.maximum(h, 0.0)
        o_ref[...] = jnp.dot(h.astype(jnp.bfloat16), w2_ref[...],
                             preferred_element_type=jnp.float32) + b2_ref[...]

    qspecs = []
    for r in range(3):
        qspecs += [pl.BlockSpec((1, bt, 32), lambda i, q=q: (q, i, 0))
                   for q in range(4)]
    return pl.pallas_call(
        body,
        grid=grid,
        in_specs=qspecs + [
            pl.BlockSpec(memory_space=pltpu.SMEM),
            pl.BlockSpec(memory_space=pltpu.SMEM),
            pl.BlockSpec(memory_space=pltpu.SMEM),
            pl.BlockSpec((bt, H), lambda i: (i, 0)),
            pl.BlockSpec((H, 2 * H), lambda i: (0, 0)),
            pl.BlockSpec((H, 2 * H), lambda i: (0, 0)),
            pl.BlockSpec((1, 2 * H), lambda i: (0, 0)),
            pl.BlockSpec((2 * H, H), lambda i: (0, 0)),
            pl.BlockSpec((1, H), lambda i: (0, 0)),
            pl.BlockSpec(memory_space=pltpu.SMEM),
        ],
        out_specs=pl.BlockSpec((bt, H), lambda i: (i, 0)),
        out_shape=jax.ShapeDtypeStruct((N_NODES, H), jnp.float32),
    )(accs[0], accs[0], accs[0], accs[0],
      accs[1], accs[1], accs[1], accs[1],
      accs[2], accs[2], accs[2], accs[2],
      scales[0], scales[1], scales[2], node_states,
      wu1a.astype(jnp.bfloat16), wu1b.astype(jnp.bfloat16),
      bu1.reshape(1, 2 * H), wu2.astype(jnp.bfloat16),
      bu2.reshape(1, H), m)


def _pad_idx(rel, epad, fill):
    pad = epad - rel.shape[0]
    return jnp.concatenate([rel, jnp.full((pad,), fill, dtype=jnp.int32)])


def kernel(node_states, relations_0, relations_1, relations_2, w0_1, b0_1,
           w0_2, b0_2, w1_1, b1_1, w1_2, b1_2, w2_1, b2_1, w2_2, b2_2, wu1,
           bu1, wu2, bu2):
    rels = (relations_0, relations_1, relations_2)
    params = ((w0_1, b0_1, w0_2, b0_2), (w1_1, b1_1, w1_2, b1_2),
              (w2_1, b2_1, w2_2, b2_2))

    ns16 = node_states.astype(jnp.bfloat16)
    g = [_gather_sc(ns16, _pad_idx(rels[r], _EPAD_G[r], 0), _EPAD_G[r])
         for r in range(3)]

    blocks = (4000, 2000, 2000)
    ys, ms = [], []
    for r in range(3):
        d = _ARITY[r] * H
        x = g[r].reshape(_EPAD_G[r] // _ARITY[r], d)
        y, m = _mlp_tc(x, params[r][0], params[r][1], params[r][2],
                       params[r][3], _E[r] // _ARITY[r], blocks[r])
        ys.append(y)
        ms.append(m)

    m_all = jnp.maximum(jnp.maximum(ms[0][0, 0], ms[1][0, 0]),
                        ms[2][0, 0]).reshape(1, 1)

    us = [_exp_tc(ys[r].reshape(_E[r], H), ms[r]) for r in range(3)]

    zeros = jnp.zeros((_ACC_ROWS, 32), dtype=jnp.float32)
    accs = [
        _scatter_sc(us[r].reshape(4 * _E[r], 32),
                    _pad_idx(rels[r], _EPAD_S[r], _SINK), zeros,
                    _EPAD_S[r], _E[r])
        for r in range(3)
    ]
    scales = [jnp.exp(8.0 * (ms[r] - m_all)) for r in range(3)]

    return _update_tc(accs, scales, node_states, wu1[:H], wu1[H:], bu1, wu2,
                      bu2, m_all)


# speedup vs baseline: 1.1518x; 1.0596x over previous
import functools

import jax
import jax.numpy as jnp
from jax import lax
from jax.experimental import pallas as pl
from jax.experimental.pallas import tpu as pltpu
from jax.experimental.pallas import tpu_sc as plsc

H = 128
N_NODES = 50000
_ARITY = (1, 2, 3)
_E = (200000, 200000, 150000)
_EPAD_G = (204800, 204800, 159744)
_EPAD_S = (204800, 204800, 155648)
_SINK = N_NODES
_ACC_ROWS = N_NODES + 48

_MESH = dict(core_axis_name="c", subcore_axis_name="s", num_cores=2,
             num_subcores=16)


_IDXB = tuple(e // 32 for e in _EPAD_G)
_IDXB_OFF = (0, _IDXB[0], _IDXB[0] + _IDXB[1])
_IDXB_TOT = sum(_IDXB)


def _gather_sc(node_states16, idx, epad):
    mesh = plsc.VectorSubcoreMesh(**_MESH)
    units = epad // (128 * 32)

    @functools.partial(
        pl.kernel,
        out_type=jax.ShapeDtypeStruct((epad, H), jnp.bfloat16),
        mesh=mesh,
        scratch_types=[
            pltpu.VMEM((2, 128), jnp.int32),
            pltpu.VMEM((2, 128, H), jnp.bfloat16),
            pltpu.SemaphoreType.DMA,
            pltpu.SemaphoreType.DMA,
            pltpu.SemaphoreType.DMA,
        ],
        compiler_params=pltpu.CompilerParams(use_tc_tiling_on_sc=False),
    )
    def k(ns_hbm, ih, gh, idx_v, rows_v, sg0, sg1, ssem):
        w = lax.axis_index("s") * 2 + lax.axis_index("c")
        row0 = w * units
        sg = (sg0, sg1)

        def load_idx(u_, b):
            pltpu.sync_copy(ih.at[pl.ds((row0 + u_) * 128, 128)],
                            idx_v.at[b])

        def fire_gather(u_, b):
            pltpu.async_copy(ns_hbm.at[idx_v.at[b]], rows_v.at[b], sg[b])

        def drain_gather(b):
            pltpu.make_async_copy(gh.at[pl.ds(0, 128)], rows_v.at[b],
                                  sg[b]).wait()

        def store(u_, b):
            pltpu.sync_copy(rows_v.at[b],
                            gh.at[pl.ds((row0 + u_) * 128, 128)])

        def unit(u_, b):
            @pl.when(u_ + 1 < units)
            def _prefetch():
                load_idx(u_ + 1, 1 - b)
                fire_gather(u_ + 1, 1 - b)

            drain_gather(b)
            store(u_, b)

        load_idx(0, 0)
        fire_gather(0, 0)

        def body(gi, _):
            unit(2 * gi, 0)
            unit(2 * gi + 1, 1)
            return 0

        lax.fori_loop(0, units // 2, body, 0)
        if units % 2:
            unit(units - 1, 0)

    return k(node_states16, idx)


def _mlp_tc(x, w1, b1, w2, b2, n_rows, block_rows):
    d = x.shape[1]
    grid = (n_rows // block_rows,)

    def body(x_ref, w1_ref, b1_ref, w2_ref, b2_ref, y_ref, mx_ref):
        h = jnp.maximum(
            jnp.dot(x_ref[...], w1_ref[...],
                    preferred_element_type=jnp.float32) + b1_ref[...], 0.0)
        y = jnp.dot(h.astype(jnp.bfloat16), w2_ref[...],
                    preferred_element_type=jnp.float32) + b2_ref[...]
        y_ref[...] = y.astype(jnp.bfloat16)

        @pl.when(pl.program_id(0) == 0)
        def _init():
            mx_ref[0, 0] = -jnp.inf

        mx_ref[0, 0] = jnp.maximum(mx_ref[0, 0], jnp.max(y))

    return pl.pallas_call(
        body,
        grid=grid,
        in_specs=[
            pl.BlockSpec((block_rows, d), lambda i: (i, 0)),
            pl.BlockSpec((d, d), lambda i: (0, 0)),
            pl.BlockSpec((1, d), lambda i: (0, 0)),
            pl.BlockSpec((d, d), lambda i: (0, 0)),
            pl.BlockSpec((1, d), lambda i: (0, 0)),
        ],
        out_specs=[
            pl.BlockSpec((block_rows, d), lambda i: (i, 0)),
            pl.BlockSpec(memory_space=pltpu.SMEM),
        ],
        out_shape=[
            jax.ShapeDtypeStruct((n_rows, d), jnp.bfloat16),
            jax.ShapeDtypeStruct((1, 1), jnp.float32),
        ],
    )(x, w1.astype(jnp.bfloat16), b1.reshape(1, d),
      w2.astype(jnp.bfloat16), b2.reshape(1, d))


def _exp_tc(y, m):
    e_rows = y.shape[0]
    bt = 2048
    grid = (pl.cdiv(e_rows, bt),)

    def body(y_ref, m_ref, u_ref):
        y32 = y_ref[...].astype(jnp.float32)
        u_ref[...] = jnp.exp(8.0 * (y32 - m_ref[0, 0]))

    return pl.pallas_call(
        body,
        grid=grid,
        in_specs=[
            pl.BlockSpec((bt, H), lambda i: (i, 0)),
            pl.BlockSpec(memory_space=pltpu.SMEM),
        ],
        out_specs=pl.BlockSpec((bt, H), lambda i: (i, 0)),
        out_shape=jax.ShapeDtypeStruct((e_rows, H), jnp.float32),
    )(y, m)


def _scatter_sc(u, s_idx, zeros, epad, e_rows):
    mesh = plsc.VectorSubcoreMesh(**_MESH)

    @functools.partial(
        pl.kernel,
        out_type=jax.ShapeDtypeStruct((4, N_NODES, 32), jnp.float32),
        mesh=mesh,
        scratch_types=[
            pltpu.VMEM((2, 256), jnp.int32),
            pltpu.VMEM((2, 256), jnp.int32),
            pltpu.VMEM((2, 256, 32), jnp.float32),
            pltpu.VMEM_SHARED((_ACC_ROWS, 32), jnp.float32),
            pltpu.SemaphoreType.DMA,
            pltpu.SemaphoreType.DMA,
            pltpu.SemaphoreType.DMA,
            pltpu.SemaphoreType.DMA,
            pltpu.SemaphoreType.DMA,
            pltpu.SemaphoreType.DMA,
        ],
        compiler_params=pltpu.CompilerParams(use_tc_tiling_on_sc=False),
    )
    def k(vh, nh, zr, out, nidx, ridx, vals, acc_sh, sn0, sn1, sg0, sg1,
          ss0, ss1):
        c = lax.axis_index("c")
        s = lax.axis_index("s")
        iota = lax.iota(jnp.int32, 16)
        sn = (sn0, sn1)
        sg = (sg0, sg1)
        ss = (ss0, ss1)
        per_tile = epad // 16
        units = per_tile // 256
        e_base = s * per_tile
        e_max = e_rows - 1
        for qi in range(2):
            q = 2 * c + qi
            pltpu.sync_copy(zr.at[pl.ds(s * 3128, 3128)],
                            acc_sh.at[pl.ds(s * 3128, 3128)])
            plsc.subcore_barrier()

            def fire_nidx(u_, b):
                pltpu.async_copy(nh.at[pl.ds(e_base + u_ * 256, 256)],
                                 nidx.at[b], sn[b])

            def fire_gather(u_, b, q=q):
                e0 = e_base + u_ * 256
                for t in range(16):
                    e_vec = jnp.minimum(e0 + t * 16 + iota, e_max)
                    ridx[b, pl.ds(t * 16, 16)] = e_vec * 4 + q
                pltpu.async_copy(vh.at[ridx.at[b]], vals.at[b], sg[b])

            def fire_scatter(b):
                pltpu.async_copy(vals.at[b], acc_sh.at[nidx.at[b]],
                                 ss[b], add=True)

            def drain_nidx(b):
                pltpu.make_async_copy(nh.at[pl.ds(0, 256)], nidx.at[b],
                                      sn[b]).wait()

            def drain_gather(b):
                pltpu.make_async_copy(vh.at[pl.ds(0, 256)], vals.at[b],
                                      sg[b]).wait()

            def drain_scatter(b):
                pltpu.make_async_copy(vals.at[b], acc_sh.at[pl.ds(0, 256)],
                                      ss[b]).wait()

            def unit(u_, b):
                @pl.when(u_ >= 2)
                def _frees():
                    drain_scatter(b)

                fire_nidx(u_, b)
                fire_gather(u_, b)

                @pl.when(u_ >= 1)
                def _flush():
                    drain_gather(1 - b)
                    drain_nidx(1 - b)
                    fire_scatter(1 - b)

            def body(gi, _):
                unit(2 * gi, 0)
                unit(2 * gi + 1, 1)
                return 0

            lax.fori_loop(0, units // 2, body, 0)
            drain_gather(1)
            drain_nidx(1)
            fire_scatter(1)
            drain_scatter(0)
            drain_scatter(1)
            plsc.subcore_barrier()
            @pl.when(s < 15)
            def _wb_full():
                pltpu.sync_copy(acc_sh.at[pl.ds(s * 3128, 3128)],
                                out.at[q, pl.ds(s * 3128, 3128)])

            @pl.when(s == 15)
            def _wb_last():
                pltpu.sync_copy(acc_sh.at[pl.ds(15 * 3128, 3080)],
                                out.at[q, pl.ds(15 * 3128, 3080)])

            plsc.subcore_barrier()

    return k(u, s_idx, zeros)


def _update_tc(accs, scales, node_states, wu1a, wu1b, bu1, wu2, bu2, m):
    bt = 1000
    grid = (N_NODES // bt,)

    def body(*refs):
        (a00, a01, a02, a03, a10, a11, a12, a13, a20, a21, a22, a23,
         s0_ref, s1_ref, s2_ref, ns_ref, w1a_ref, w1b_ref, b1_ref, w2_ref,
         b2_ref, m_ref, o_ref) = refs
        aq = ((a00, a10, a20), (a01, a11, a21), (a02, a12, a22),
              (a03, a13, a23))
        sc = (s0_ref[0, 0], s1_ref[0, 0], s2_ref[0, 0])
        h = jnp.dot(ns_ref[...].astype(jnp.bfloat16), w1b_ref[...],
                    preferred_element_type=jnp.float32) + b1_ref[...]
        for q in range(4):
            tot = (sc[0] * aq[q][0][0] + sc[1] * aq[q][1][0]
                   + sc[2] * aq[q][2][0])
            t = jnp.log(tot + 1e-16) * 0.125 + m_ref[0, 0]
            h += jnp.dot(t.astype(jnp.bfloat16),
                         w1a_ref[pl.ds(q * 32, 32), :],
                         preferred_element_type=jnp.float32)
        h = jnp.maximum(h, 0.0)
        o_ref[...] = jnp.dot(h.astype(jnp.bfloat16), w2_ref[...],
                             preferred_element_type=jnp.float32) + b2_ref[...]

    qspecs = []
    for r in range(3):
        qspecs += [pl.BlockSpec((1, bt, 32), lambda i, q=q: (q, i, 0))
                   for q in range(4)]
    return pl.pallas_call(
        body,
        grid=grid,
        in_specs=qspecs + [
            pl.BlockSpec(memory_space=pltpu.SMEM),
            pl.BlockSpec(memory_space=pltpu.SMEM),
            pl.BlockSpec(memory_space=pltpu.SMEM),
            pl.BlockSpec((bt, H), lambda i: (i, 0)),
            pl.BlockSpec((H, 2 * H), lambda i: (0, 0)),
            pl.BlockSpec((H, 2 * H), lambda i: (0, 0)),
            pl.BlockSpec((1, 2 * H), lambda i: (0, 0)),
            pl.BlockSpec((2 * H, H), lambda i: (0, 0)),
            pl.BlockSpec((1, H), lambda i: (0, 0)),
            pl.BlockSpec(memory_space=pltpu.SMEM),
        ],
        out_specs=pl.BlockSpec((bt, H), lambda i: (i, 0)),
        out_shape=jax.ShapeDtypeStruct((N_NODES, H), jnp.float32),
    )(accs[0], accs[0], accs[0], accs[0],
      accs[1], accs[1], accs[1], accs[1],
      accs[2], accs[2], accs[2], accs[2],
      scales[0], scales[1], scales[2], node_states,
      wu1a.astype(jnp.bfloat16), wu1b.astype(jnp.bfloat16),
      bu1.reshape(1, 2 * H), wu2.astype(jnp.bfloat16),
      bu2.reshape(1, H), m)


def _pad_idx(rel, epad, fill):
    pad = epad - rel.shape[0]
    return jnp.concatenate([rel, jnp.full((pad,), fill, dtype=jnp.int32)])


def kernel(node_states, relations_0, relations_1, relations_2, w0_1, b0_1,
           w0_2, b0_2, w1_1, b1_1, w1_2, b1_2, w2_1, b2_1, w2_2, b2_2, wu1,
           bu1, wu2, bu2):
    rels = (relations_0, relations_1, relations_2)
    params = ((w0_1, b0_1, w0_2, b0_2), (w1_1, b1_1, w1_2, b1_2),
              (w2_1, b2_1, w2_2, b2_2))

    ns16 = node_states.astype(jnp.bfloat16)
    g = [_gather_sc(ns16, _pad_idx(rels[r], _EPAD_G[r], 0), _EPAD_G[r])
         for r in range(3)]

    blocks = (8000, 4000, 2000)
    ys, ms = [], []
    for r in range(3):
        d = _ARITY[r] * H
        x = g[r].reshape(_EPAD_G[r] // _ARITY[r], d)
        y, m = _mlp_tc(x, params[r][0], params[r][1], params[r][2],
                       params[r][3], _E[r] // _ARITY[r], blocks[r])
        ys.append(y)
        ms.append(m)

    m_all = jnp.maximum(jnp.maximum(ms[0][0, 0], ms[1][0, 0]),
                        ms[2][0, 0]).reshape(1, 1)

    us = [_exp_tc(ys[r].reshape(_E[r], H), ms[r]) for r in range(3)]

    zeros = jnp.zeros((_ACC_ROWS, 32), dtype=jnp.float32)
    accs = [
        _scatter_sc(us[r].reshape(4 * _E[r], 32),
                    _pad_idx(rels[r], _EPAD_S[r], _SINK), zeros,
                    _EPAD_S[r], _E[r])
        for r in range(3)
    ]
    scales = [jnp.exp(8.0 * (ms[r] - m_all)) for r in range(3)]

    return _update_tc(accs, scales, node_states, wu1[:H], wu1[H:], bu1, wu2,
                      bu2, m_all)
